# baseline (speedup 1.0000x reference)
import functools

import jax
import jax.numpy as jnp
from jax import lax
from jax.experimental import pallas as pl
from jax.experimental.pallas import tpu as pltpu
from jax.experimental.pallas import tpu_sc as plsc

TOP_K = 2
BLK_M = 128
TOK_BLK = 256
LANE = 128


def _round_up(v, m):
    return (v + m - 1) // m * m


def _sigmoid(v):
    return 1.0 / (1.0 + jnp.exp(-v))


def _gather_rows(table, idx):
    V, D = table.shape
    (B,) = idx.shape
    info = plsc.get_sparse_core_info()
    nw = info.num_cores * info.num_subcores
    assert D % info.num_lanes == 0 and B % (8 * nw) == 0
    b_per_w = B // nw
    mesh = plsc.VectorSubcoreMesh(core_axis_name="c", subcore_axis_name="s")

    @functools.partial(
        pl.kernel,
        mesh=mesh,
        out_type=jax.ShapeDtypeStruct((B, D), jnp.float32),
        scratch_types=[
            pltpu.VMEM((b_per_w,), jnp.int32),
            pltpu.VMEM((b_per_w, D), jnp.float32),
            pltpu.SemaphoreType.DMA,
        ],
    )
    def k(table_hbm, idx_hbm, out_hbm, idx_v, rows_v, sem):
        wid = lax.axis_index("s") * info.num_cores + lax.axis_index("c")
        base = wid * b_per_w
        pltpu.sync_copy(idx_hbm.at[pl.ds(base, b_per_w)], idx_v)
        pltpu.async_copy(table_hbm.at[idx_v], rows_v, sem).wait()
        pltpu.sync_copy(rows_v, out_hbm.at[pl.ds(base, b_per_w)])

    return k(table, idx)


def kernel(x, Wg, W1, W3, W2, Ws1, Ws3, Ws2):
    Bb, Tt, C = x.shape
    S = Bb * Tt
    E = Wg.shape[1]
    H = W1.shape[2]
    H_PAD = _round_up(H, 3 * LANE)
    N_BUF = S * TOP_K + E * BLK_M
    N_BLK = N_BUF // BLK_M
    f32 = jnp.float32

    x_flat = x.reshape(S, C)

    wg_pad = jnp.pad(Wg, ((0, 0), (0, LANE - E)))

    def _router_body(x_ref, wg_ref, w_ref, i_ref, aux_ref):
        xb = x_ref[...]
        logits = jnp.dot(xb, wg_ref[...], preferred_element_type=f32)
        col = lax.broadcasted_iota(jnp.int32, logits.shape, 1)
        valid = col < E
        ml = jnp.where(valid, logits, -1e30)
        m = jnp.max(ml, axis=1, keepdims=True)
        p = jnp.where(valid, jnp.exp(ml - m), 0.0)
        rw = p / jnp.sum(p, axis=1, keepdims=True)
        w0 = jnp.max(rw, axis=1, keepdims=True)
        is0 = jnp.logical_and(rw == w0, valid)
        e0 = jnp.min(jnp.where(is0, col, E), axis=1, keepdims=True)
        rwm = jnp.where(col == e0, -1.0, rw)
        w1v = jnp.max(rwm, axis=1, keepdims=True)
        is1 = jnp.logical_and(rwm == w1v,
                              jnp.logical_and(valid, col != e0))
        e1 = jnp.min(jnp.where(is1, col, E), axis=1, keepdims=True)
        imp = jnp.sum(rw, axis=0, keepdims=True)
        ld = jnp.sum(jnp.where(col == e0, 1.0, 0.0), axis=0, keepdims=True)
        aux = E * jnp.sum(imp * ld) / (S * S)
        aux_ref[...] = jnp.full((8, LANE), aux, dtype=f32)
        sw = w0 + w1v
        w_ref[...] = jnp.where(col == 0, w0 / sw,
                               jnp.where(col == 1, w1v / sw, 0.0))
        i_ref[...] = jnp.where(col == 0, e0,
                               jnp.where(col == 1, e1, 0)).astype(jnp.int32)

    w_pad, i_pad, aux2 = pl.pallas_call(
        _router_body,
        out_shape=[
            jax.ShapeDtypeStruct((S, LANE), f32),
            jax.ShapeDtypeStruct((S, LANE), jnp.int32),
            jax.ShapeDtypeStruct((8, LANE), f32),
        ],
    )(x_flat, wg_pad)
    aux_loss = aux2[0, 0]

    e0 = i_pad[:, 0]
    e1 = i_pad[:, 1]
    oh = (jax.nn.one_hot(e0, E, dtype=jnp.int32)
          + jax.nn.one_hot(e1, E, dtype=jnp.int32))
    cum = jnp.cumsum(oh, axis=0)
    excl = cum - oh
    rank0 = jnp.take_along_axis(excl, e0[:, None], axis=1)[:, 0]
    rank1 = jnp.take_along_axis(excl, e1[:, None], axis=1)[:, 0]
    counts = cum[-1]
    blk_per_e = (counts + BLK_M - 1) // BLK_M
    ends = jnp.cumsum(blk_per_e)
    starts_rows = (ends - blk_per_e) * BLK_M
    pos0 = (starts_rows[e0] + rank0).astype(jnp.int32)
    pos1 = (starts_rows[e1] + rank1).astype(jnp.int32)
    tid = jnp.arange(S, dtype=jnp.int32)
    sorted_tid = (jnp.zeros((N_BUF,), jnp.int32)
                  .at[pos0].set(tid).at[pos1].set(tid))
    blk_expert = jnp.minimum(
        jnp.searchsorted(ends, jnp.arange(N_BLK, dtype=jnp.int32),
                         side="right"),
        E - 1).astype(jnp.int32)

    x_sorted = _gather_rows(x_flat, sorted_tid)

    w1p = jnp.pad(W1, ((0, 0), (0, 0), (0, H_PAD - H)))
    w3p = jnp.pad(W3, ((0, 0), (0, 0), (0, H_PAD - H)))
    w2p = jnp.pad(W2, ((0, 0), (0, H_PAD - H), (0, 0)))

    def _expert_body(be_ref, xs_ref, w1_ref, w3_ref, w2_ref, o_ref):
        xb = xs_ref[...]
        h1 = jnp.dot(xb, w1_ref[0], preferred_element_type=f32)
        h3 = jnp.dot(xb, w3_ref[0], preferred_element_type=f32)
        glu = h1 * _sigmoid(h1) * h3
        o_ref[...] = jnp.dot(glu, w2_ref[0], preferred_element_type=f32)

    grid_spec = pltpu.PrefetchScalarGridSpec(
        num_scalar_prefetch=1,
        grid=(N_BLK,),
        in_specs=[
            pl.BlockSpec((BLK_M, C), lambda m, be: (m, 0)),
            pl.BlockSpec((1, C, H_PAD), lambda m, be: (be[m], 0, 0)),
            pl.BlockSpec((1, C, H_PAD), lambda m, be: (be[m], 0, 0)),
            pl.BlockSpec((1, H_PAD, C), lambda m, be: (be[m], 0, 0)),
        ],
        out_specs=pl.BlockSpec((BLK_M, C), lambda m, be: (m, 0)),
    )
    y_sorted = pl.pallas_call(
        _expert_body,
        grid_spec=grid_spec,
        out_shape=jax.ShapeDtypeStruct((N_BUF, C), f32),
    )(blk_expert, x_sorted, w1p, w3p, w2p)

    pos01 = jnp.concatenate([pos0, pos1])
    g = _gather_rows(y_sorted, pos01)
    g0 = g[:S]
    g1 = g[S:]

    ws1p = jnp.pad(Ws1, ((0, 0), (0, H_PAD - H)))
    ws3p = jnp.pad(Ws3, ((0, 0), (0, H_PAD - H)))
    ws2p = jnp.pad(Ws2, ((0, H_PAD - H), (0, 0)))

    def _shared_body(x_ref, ws1_ref, ws3_ref, ws2_ref,
                     g0_ref, g1_ref, wf_ref, o_ref):
        xb = x_ref[...]
        h1 = jnp.dot(xb, ws1_ref[...], preferred_element_type=f32)
        h3 = jnp.dot(xb, ws3_ref[...], preferred_element_type=f32)
        glu = h1 * _sigmoid(h1) * h3
        sh = jnp.dot(glu, ws2_ref[...], preferred_element_type=f32)
        wf = wf_ref[...]
        o_ref[...] = (sh + wf[:, 0:1] * g0_ref[...]
                      + wf[:, 1:2] * g1_ref[...])

    out = pl.pallas_call(
        _shared_body,
        grid=(S // TOK_BLK,),
        in_specs=[
            pl.BlockSpec((TOK_BLK, C), lambda m: (m, 0)),
            pl.BlockSpec((C, H_PAD), lambda m: (0, 0)),
            pl.BlockSpec((C, H_PAD), lambda m: (0, 0)),
            pl.BlockSpec((H_PAD, C), lambda m: (0, 0)),
            pl.BlockSpec((TOK_BLK, C), lambda m: (m, 0)),
            pl.BlockSpec((TOK_BLK, C), lambda m: (m, 0)),
            pl.BlockSpec((TOK_BLK, LANE), lambda m: (m, 0)),
        ],
        out_specs=pl.BlockSpec((TOK_BLK, C), lambda m: (m, 0)),
        out_shape=jax.ShapeDtypeStruct((S, C), f32),
    )(x_flat, ws1p, ws3p, ws2p, g0, g1, w_pad)

    return out.reshape(Bb, Tt, C), aux_loss


# speedup vs baseline: 1.2156x; 1.2156x over previous
import functools

import jax
import jax.numpy as jnp
from jax import lax
from jax.experimental import pallas as pl
from jax.experimental.pallas import tpu as pltpu
from jax.experimental.pallas import tpu_sc as plsc

TOP_K = 2
BLK_M = 128
TOK_BLK = 256
LANE = 128


def _sigmoid(v):
    return 1.0 / (1.0 + jnp.exp(-v))


def _gather_rows(table, idx):
    V, D = table.shape
    (B,) = idx.shape
    info = plsc.get_sparse_core_info()
    nw = info.num_cores * info.num_subcores
    assert D % info.num_lanes == 0 and B % (8 * nw) == 0
    b_per_w = B // nw
    mesh = plsc.VectorSubcoreMesh(core_axis_name="c", subcore_axis_name="s")

    @functools.partial(
        pl.kernel,
        mesh=mesh,
        out_type=jax.ShapeDtypeStruct((B, D), jnp.float32),
        scratch_types=[
            pltpu.VMEM((b_per_w,), jnp.int32),
            pltpu.VMEM((b_per_w, D), jnp.float32),
            pltpu.SemaphoreType.DMA,
        ],
    )
    def k(table_hbm, idx_hbm, out_hbm, idx_v, rows_v, sem):
        wid = lax.axis_index("s") * info.num_cores + lax.axis_index("c")
        base = wid * b_per_w
        pltpu.sync_copy(idx_hbm.at[pl.ds(base, b_per_w)], idx_v)
        pltpu.async_copy(table_hbm.at[idx_v], rows_v, sem).wait()
        pltpu.sync_copy(rows_v, out_hbm.at[pl.ds(base, b_per_w)])

    return k(table, idx)


def kernel(x, Wg, W1, W3, W2, Ws1, Ws3, Ws2):
    Bb, Tt, C = x.shape
    S = Bb * Tt
    E = Wg.shape[1]
    H = W1.shape[2]
    N_BUF = S * TOP_K + E * BLK_M
    N_BLK = N_BUF // BLK_M
    f32 = jnp.float32

    x_flat = x.reshape(S, C)

    wg_pad = jnp.pad(Wg, ((0, 0), (0, LANE - E)))

    def _router_body(x_ref, wg_ref, w_ref, i_ref, aux_ref):
        xb = x_ref[...]
        logits = jnp.dot(xb, wg_ref[...], preferred_element_type=f32)
        col = lax.broadcasted_iota(jnp.int32, logits.shape, 1)
        valid = col < E
        ml = jnp.where(valid, logits, -1e30)
        m = jnp.max(ml, axis=1, keepdims=True)
        p = jnp.where(valid, jnp.exp(ml - m), 0.0)
        rw = p / jnp.sum(p, axis=1, keepdims=True)
        w0 = jnp.max(rw, axis=1, keepdims=True)
        is0 = jnp.logical_and(rw == w0, valid)
        e0 = jnp.min(jnp.where(is0, col, E), axis=1, keepdims=True)
        rwm = jnp.where(col == e0, -1.0, rw)
        w1v = jnp.max(rwm, axis=1, keepdims=True)
        is1 = jnp.logical_and(rwm == w1v,
                              jnp.logical_and(valid, col != e0))
        e1 = jnp.min(jnp.where(is1, col, E), axis=1, keepdims=True)
        imp = jnp.sum(rw, axis=0, keepdims=True)
        ld = jnp.sum(jnp.where(col == e0, 1.0, 0.0), axis=0, keepdims=True)
        aux = E * jnp.sum(imp * ld) / (S * S)
        aux_ref[...] = jnp.full((8, LANE), aux, dtype=f32)
        sw = w0 + w1v
        w_ref[...] = jnp.where(col == 0, w0 / sw,
                               jnp.where(col == 1, w1v / sw, 0.0))
        i_ref[...] = jnp.where(col == 0, e0,
                               jnp.where(col == 1, e1, 0)).astype(jnp.int32)

    w_pad, i_pad, aux2 = pl.pallas_call(
        _router_body,
        out_shape=[
            jax.ShapeDtypeStruct((S, LANE), f32),
            jax.ShapeDtypeStruct((S, LANE), jnp.int32),
            jax.ShapeDtypeStruct((8, LANE), f32),
        ],
    )(x_flat, wg_pad)
    aux_loss = aux2[0, 0]

    e0 = i_pad[:, 0]
    e1 = i_pad[:, 1]
    oh = (jax.nn.one_hot(e0, E, dtype=jnp.int32)
          + jax.nn.one_hot(e1, E, dtype=jnp.int32))
    cum = jnp.cumsum(oh, axis=0)
    excl = cum - oh
    rank0 = jnp.take_along_axis(excl, e0[:, None], axis=1)[:, 0]
    rank1 = jnp.take_along_axis(excl, e1[:, None], axis=1)[:, 0]
    counts = cum[-1]
    blk_per_e = (counts + BLK_M - 1) // BLK_M
    ends = jnp.cumsum(blk_per_e)
    starts_rows = (ends - blk_per_e) * BLK_M
    pos0 = (starts_rows[e0] + rank0).astype(jnp.int32)
    pos1 = (starts_rows[e1] + rank1).astype(jnp.int32)
    tid = jnp.arange(S, dtype=jnp.int32)
    sorted_tid = (jnp.zeros((N_BUF,), jnp.int32)
                  .at[pos0].set(tid).at[pos1].set(tid))
    blk_expert = jnp.minimum(
        jnp.searchsorted(ends, jnp.arange(N_BLK, dtype=jnp.int32),
                         side="right"),
        E - 1).astype(jnp.int32)

    x_sorted = _gather_rows(x_flat, sorted_tid)

    def _expert_body(be_ref, xs_ref, w1_ref, w3_ref, w2_ref, o_ref):
        xb = xs_ref[...]
        h1 = jnp.dot(xb, w1_ref[0], preferred_element_type=f32)
        h3 = jnp.dot(xb, w3_ref[0], preferred_element_type=f32)
        glu = h1 * _sigmoid(h1) * h3
        o_ref[...] = jnp.dot(glu, w2_ref[0], preferred_element_type=f32)

    grid_spec = pltpu.PrefetchScalarGridSpec(
        num_scalar_prefetch=1,
        grid=(N_BLK,),
        in_specs=[
            pl.BlockSpec((BLK_M, C), lambda m, be: (m, 0)),
            pl.BlockSpec((1, C, H), lambda m, be: (be[m], 0, 0)),
            pl.BlockSpec((1, C, H), lambda m, be: (be[m], 0, 0)),
            pl.BlockSpec((1, H, C), lambda m, be: (be[m], 0, 0)),
        ],
        out_specs=pl.BlockSpec((BLK_M, C), lambda m, be: (m, 0)),
    )
    y_sorted = pl.pallas_call(
        _expert_body,
        grid_spec=grid_spec,
        out_shape=jax.ShapeDtypeStruct((N_BUF, C), f32),
    )(blk_expert, x_sorted, W1, W3, W2)

    pos01 = jnp.concatenate([pos0, pos1])
    g = _gather_rows(y_sorted, pos01)

    def _shared_body(x_ref, ws1_ref, ws3_ref, ws2_ref,
                     g0_ref, g1_ref, wf_ref, o_ref):
        xb = x_ref[...]
        h1 = jnp.dot(xb, ws1_ref[...], preferred_element_type=f32)
        h3 = jnp.dot(xb, ws3_ref[...], preferred_element_type=f32)
        glu = h1 * _sigmoid(h1) * h3
        sh = jnp.dot(glu, ws2_ref[...], preferred_element_type=f32)
        wf = wf_ref[...]
        o_ref[...] = (sh + wf[:, 0:1] * g0_ref[...]
                      + wf[:, 1:2] * g1_ref[...])

    out = pl.pallas_call(
        _shared_body,
        grid=(S // TOK_BLK,),
        in_specs=[
            pl.BlockSpec((TOK_BLK, C), lambda m: (m, 0)),
            pl.BlockSpec((C, H), lambda m: (0, 0)),
            pl.BlockSpec((C, H), lambda m: (0, 0)),
            pl.BlockSpec((H, C), lambda m: (0, 0)),
            pl.BlockSpec((TOK_BLK, C), lambda m: (m, 0)),
            pl.BlockSpec((TOK_BLK, C), lambda m: (m + S // TOK_BLK, 0)),
            pl.BlockSpec((TOK_BLK, LANE), lambda m: (m, 0)),
        ],
        out_specs=pl.BlockSpec((TOK_BLK, C), lambda m: (m, 0)),
        out_shape=jax.ShapeDtypeStruct((S, C), f32),
    )(x_flat, Ws1, Ws3, Ws2, g, g, w_pad)

    return out.reshape(Bb, Tt, C), aux_loss


# speedup vs baseline: 1.3339x; 1.0973x over previous
import functools

import jax
import jax.numpy as jnp
from jax import lax
from jax.experimental import pallas as pl
from jax.experimental.pallas import tpu as pltpu
from jax.experimental.pallas import tpu_sc as plsc

TOP_K = 2
BLK_M = 128
TOK_BLK = 256
LANE = 128


def _sigmoid(v):
    return 1.0 / (1.0 + jnp.exp(-v))


def _gather_rows(table, idx):
    V, D = table.shape
    (B,) = idx.shape
    info = plsc.get_sparse_core_info()
    nw = info.num_cores * info.num_subcores
    assert D % info.num_lanes == 0 and B % (8 * nw) == 0
    b_per_w = B // nw
    mesh = plsc.VectorSubcoreMesh(core_axis_name="c", subcore_axis_name="s")

    @functools.partial(
        pl.kernel,
        mesh=mesh,
        out_type=jax.ShapeDtypeStruct((B, D), jnp.float32),
        scratch_types=[
            pltpu.VMEM((b_per_w,), jnp.int32),
            pltpu.VMEM((b_per_w, D), jnp.float32),
            pltpu.SemaphoreType.DMA,
        ],
    )
    def k(table_hbm, idx_hbm, out_hbm, idx_v, rows_v, sem):
        wid = lax.axis_index("s") * info.num_cores + lax.axis_index("c")
        base = wid * b_per_w
        pltpu.sync_copy(idx_hbm.at[pl.ds(base, b_per_w)], idx_v)
        pltpu.async_copy(table_hbm.at[idx_v], rows_v, sem).wait()
        pltpu.sync_copy(rows_v, out_hbm.at[pl.ds(base, b_per_w)])

    return k(table, idx)


def kernel(x, Wg, W1, W3, W2, Ws1, Ws3, Ws2):
    Bb, Tt, C = x.shape
    S = Bb * Tt
    E = Wg.shape[1]
    H = W1.shape[2]
    N_BUF = S * TOP_K + E * BLK_M
    N_BLK = N_BUF // BLK_M
    f32 = jnp.float32

    x_flat = x.reshape(S, C)

    wg_pad = jnp.pad(Wg, ((0, 0), (0, LANE - E)))

    def _router_body(x_ref, wg_ref, w_ref, i_ref, aux_ref):
        xb = x_ref[...]
        logits = jnp.dot(xb, wg_ref[...], preferred_element_type=f32)
        col = lax.broadcasted_iota(jnp.int32, logits.shape, 1)
        valid = col < E
        ml = jnp.where(valid, logits, -1e30)
        m = jnp.max(ml, axis=1, keepdims=True)
        p = jnp.where(valid, jnp.exp(ml - m), 0.0)
        rw = p / jnp.sum(p, axis=1, keepdims=True)
        w0 = jnp.max(rw, axis=1, keepdims=True)
        is0 = jnp.logical_and(rw == w0, valid)
        e0 = jnp.min(jnp.where(is0, col, E), axis=1, keepdims=True)
        rwm = jnp.where(col == e0, -1.0, rw)
        w1v = jnp.max(rwm, axis=1, keepdims=True)
        is1 = jnp.logical_and(rwm == w1v,
                              jnp.logical_and(valid, col != e0))
        e1 = jnp.min(jnp.where(is1, col, E), axis=1, keepdims=True)
        imp = jnp.sum(rw, axis=0, keepdims=True)
        ld = jnp.sum(jnp.where(col == e0, 1.0, 0.0), axis=0, keepdims=True)
        aux = E * jnp.sum(imp * ld) / (S * S)
        aux_ref[...] = jnp.full((8, LANE), aux, dtype=f32)
        sw = w0 + w1v
        w_ref[...] = jnp.where(col == 0, w0 / sw,
                               jnp.where(col == 1, w1v / sw, 0.0))
        i_ref[...] = jnp.where(col == 0, e0,
                               jnp.where(col == 1, e1, 0)).astype(jnp.int32)

    w_pad, i_pad, aux2 = pl.pallas_call(
        _router_body,
        out_shape=[
            jax.ShapeDtypeStruct((S, LANE), f32),
            jax.ShapeDtypeStruct((S, LANE), jnp.int32),
            jax.ShapeDtypeStruct((8, LANE), f32),
        ],
    )(x_flat, wg_pad)
    aux_loss = aux2[0, 0]

    e0 = i_pad[:, 0]
    e1 = i_pad[:, 1]
    oh = (jax.nn.one_hot(e0, E, dtype=jnp.int32)
          + jax.nn.one_hot(e1, E, dtype=jnp.int32))
    cum = jnp.cumsum(oh, axis=0)
    excl = cum - oh
    rank0 = jnp.take_along_axis(excl, e0[:, None], axis=1)[:, 0]
    rank1 = jnp.take_along_axis(excl, e1[:, None], axis=1)[:, 0]
    counts = cum[-1]
    blk_per_e = (counts + BLK_M - 1) // BLK_M
    ends = jnp.cumsum(blk_per_e)
    starts_rows = (ends - blk_per_e) * BLK_M
    pos0 = (starts_rows[e0] + rank0).astype(jnp.int32)
    pos1 = (starts_rows[e1] + rank1).astype(jnp.int32)
    tid = jnp.arange(S, dtype=jnp.int32)
    base_tid = jnp.arange(N_BUF, dtype=jnp.int32) % S
    sorted_tid = base_tid.at[pos0].set(tid).at[pos1].set(tid)
    blk_expert = jnp.minimum(
        jnp.searchsorted(ends, jnp.arange(N_BLK, dtype=jnp.int32),
                         side="right"),
        E - 1).astype(jnp.int32)

    x_sorted = _gather_rows(x_flat, sorted_tid)

    def _expert_body(be_ref, xs_ref, w1_ref, w3_ref, w2_ref, o_ref):
        xb = xs_ref[...]
        h1 = jnp.dot(xb, w1_ref[0], preferred_element_type=f32)
        h3 = jnp.dot(xb, w3_ref[0], preferred_element_type=f32)
        glu = h1 * _sigmoid(h1) * h3
        o_ref[...] = jnp.dot(glu, w2_ref[0], preferred_element_type=f32)

    grid_spec = pltpu.PrefetchScalarGridSpec(
        num_scalar_prefetch=1,
        grid=(N_BLK,),
        in_specs=[
            pl.BlockSpec((BLK_M, C), lambda m, be: (m, 0)),
            pl.BlockSpec((1, C, H), lambda m, be: (be[m], 0, 0)),
            pl.BlockSpec((1, C, H), lambda m, be: (be[m], 0, 0)),
            pl.BlockSpec((1, H, C), lambda m, be: (be[m], 0, 0)),
        ],
        out_specs=pl.BlockSpec((BLK_M, C), lambda m, be: (m, 0)),
    )
    y_sorted = pl.pallas_call(
        _expert_body,
        grid_spec=grid_spec,
        out_shape=jax.ShapeDtypeStruct((N_BUF, C), f32),
    )(blk_expert, x_sorted, W1, W3, W2)

    pos01 = jnp.concatenate([pos0, pos1])
    g = _gather_rows(y_sorted, pos01)

    def _shared_body(x_ref, ws1_ref, ws3_ref, ws2_ref, o_ref):
        xb = x_ref[...]
        h1 = jnp.dot(xb, ws1_ref[...], preferred_element_type=f32)
        h3 = jnp.dot(xb, ws3_ref[...], preferred_element_type=f32)
        glu = h1 * _sigmoid(h1) * h3
        o_ref[...] = jnp.dot(glu, ws2_ref[...], preferred_element_type=f32)

    shared_out = pl.pallas_call(
        _shared_body,
        grid=(S // TOK_BLK,),
        in_specs=[
            pl.BlockSpec((TOK_BLK, C), lambda m: (m, 0)),
            pl.BlockSpec((C, H), lambda m: (0, 0)),
            pl.BlockSpec((C, H), lambda m: (0, 0)),
            pl.BlockSpec((H, C), lambda m: (0, 0)),
        ],
        out_specs=pl.BlockSpec((TOK_BLK, C), lambda m: (m, 0)),
        out_shape=jax.ShapeDtypeStruct((S, C), f32),
    )(x_flat, Ws1, Ws3, Ws2)

    def _combine_body(sh_ref, g0_ref, g1_ref, wf_ref, o_ref):
        wf = wf_ref[...]
        o_ref[...] = (sh_ref[...] + wf[:, 0:1] * g0_ref[...]
                      + wf[:, 1:2] * g1_ref[...])

    out = pl.pallas_call(
        _combine_body,
        grid=(S // TOK_BLK,),
        in_specs=[
            pl.BlockSpec((TOK_BLK, C), lambda m: (m, 0)),
            pl.BlockSpec((TOK_BLK, C), lambda m: (m, 0)),
            pl.BlockSpec((TOK_BLK, C), lambda m: (m + S // TOK_BLK, 0)),
            pl.BlockSpec((TOK_BLK, LANE), lambda m: (m, 0)),
        ],
        out_specs=pl.BlockSpec((TOK_BLK, C), lambda m: (m, 0)),
        out_shape=jax.ShapeDtypeStruct((S, C), f32),
    )(shared_out, g, g, w_pad)

    return out.reshape(Bb, Tt, C), aux_loss


# speedup vs baseline: 1.5155x; 1.1362x over previous
import functools

import jax
import jax.numpy as jnp
from jax import lax
from jax.experimental import pallas as pl
from jax.experimental.pallas import tpu as pltpu
from jax.experimental.pallas import tpu_sc as plsc

TOP_K = 2
BLK_M = 128
TOK_BLK = 256
LANE = 128


def _sigmoid(v):
    return 1.0 / (1.0 + jnp.exp(-v))


def _gather_rows(table, idx):
    V, D = table.shape
    (B,) = idx.shape
    info = plsc.get_sparse_core_info()
    nw = info.num_cores * info.num_subcores
    assert D % info.num_lanes == 0 and B % (8 * nw) == 0
    b_per_w = B // nw
    mesh = plsc.VectorSubcoreMesh(core_axis_name="c", subcore_axis_name="s")

    @functools.partial(
        pl.kernel,
        mesh=mesh,
        out_type=jax.ShapeDtypeStruct((B, D), jnp.float32),
        scratch_types=[
            pltpu.VMEM((b_per_w,), jnp.int32),
            pltpu.VMEM((b_per_w, D), jnp.float32),
            pltpu.SemaphoreType.DMA,
        ],
    )
    def k(table_hbm, idx_hbm, out_hbm, idx_v, rows_v, sem):
        wid = lax.axis_index("s") * info.num_cores + lax.axis_index("c")
        base = wid * b_per_w
        pltpu.sync_copy(idx_hbm.at[pl.ds(base, b_per_w)], idx_v)
        pltpu.async_copy(table_hbm.at[idx_v], rows_v, sem).wait()
        pltpu.sync_copy(rows_v, out_hbm.at[pl.ds(base, b_per_w)])

    return k(table, idx)


def kernel(x, Wg, W1, W3, W2, Ws1, Ws3, Ws2):
    Bb, Tt, C = x.shape
    S = Bb * Tt
    E = Wg.shape[1]
    H = W1.shape[2]
    N_BUF = S * TOP_K + E * BLK_M
    N_BLK = N_BUF // BLK_M
    f32 = jnp.float32

    x_flat = x.reshape(S, C)

    wg_pad = jnp.pad(Wg, ((0, 0), (0, LANE - E)))

    def _router_body(x_ref, wg_ref, w_ref, i_ref, aux_ref):
        xb = x_ref[...]
        logits = jnp.dot(xb, wg_ref[...], preferred_element_type=f32)
        col = lax.broadcasted_iota(jnp.int32, logits.shape, 1)
        valid = col < E
        ml = jnp.where(valid, logits, -1e30)
        m = jnp.max(ml, axis=1, keepdims=True)
        p = jnp.where(valid, jnp.exp(ml - m), 0.0)
        rw = p / jnp.sum(p, axis=1, keepdims=True)
        w0 = jnp.max(rw, axis=1, keepdims=True)
        is0 = jnp.logical_and(rw == w0, valid)
        e0 = jnp.min(jnp.where(is0, col, E), axis=1, keepdims=True)
        rwm = jnp.where(col == e0, -1.0, rw)
        w1v = jnp.max(rwm, axis=1, keepdims=True)
        is1 = jnp.logical_and(rwm == w1v,
                              jnp.logical_and(valid, col != e0))
        e1 = jnp.min(jnp.where(is1, col, E), axis=1, keepdims=True)
        imp = jnp.sum(rw, axis=0, keepdims=True)
        ld = jnp.sum(jnp.where(col == e0, 1.0, 0.0), axis=0, keepdims=True)
        aux = E * jnp.sum(imp * ld) / (S * S)
        aux_ref[...] = jnp.full((8, LANE), aux, dtype=f32)
        sw = w0 + w1v
        w_ref[...] = jnp.where(col == 0, w0 / sw,
                               jnp.where(col == 1, w1v / sw, 0.0))
        i_ref[...] = jnp.where(col == 0, e0,
                               jnp.where(col == 1, e1, 0)).astype(jnp.int32)

    w_pad, i_pad, aux2 = pl.pallas_call(
        _router_body,
        out_shape=[
            jax.ShapeDtypeStruct((S, LANE), f32),
            jax.ShapeDtypeStruct((S, LANE), jnp.int32),
            jax.ShapeDtypeStruct((8, LANE), f32),
        ],
    )(x_flat, wg_pad)
    aux_loss = aux2[0, 0]

    e0 = i_pad[:, 0]
    e1 = i_pad[:, 1]
    earange = jnp.arange(E, dtype=jnp.int32)
    oh0 = (e0[:, None] == earange[None, :]).astype(jnp.int32)
    oh1 = (e1[:, None] == earange[None, :]).astype(jnp.int32)
    oh = oh0 + oh1
    cum = jnp.cumsum(oh, axis=0)
    excl = cum - oh
    counts = cum[-1]
    blk_per_e = (counts + BLK_M - 1) // BLK_M
    ends = jnp.cumsum(blk_per_e)
    starts_rows = (ends - blk_per_e) * BLK_M
    slot0 = starts_rows[None, :] + excl
    pos0 = jnp.sum(slot0 * oh0, axis=1).astype(jnp.int32)
    pos1 = jnp.sum(slot0 * oh1, axis=1).astype(jnp.int32)
    tid = jnp.arange(S, dtype=jnp.int32)
    base_tid = jnp.arange(N_BUF, dtype=jnp.int32) % S
    sorted_tid = base_tid.at[pos0].set(tid).at[pos1].set(tid)
    blk_arange = jnp.arange(N_BLK, dtype=jnp.int32)
    blk_expert = jnp.minimum(
        jnp.sum((ends[None, :] <= blk_arange[:, None]).astype(jnp.int32),
                axis=1),
        E - 1).astype(jnp.int32)

    x_sorted = _gather_rows(x_flat, sorted_tid)

    def _expert_body(be_ref, xs_ref, w1_ref, w3_ref, w2_ref, o_ref):
        xb = xs_ref[...]
        h1 = jnp.dot(xb, w1_ref[0], preferred_element_type=f32)
        h3 = jnp.dot(xb, w3_ref[0], preferred_element_type=f32)
        glu = h1 * _sigmoid(h1) * h3
        o_ref[...] = jnp.dot(glu, w2_ref[0], preferred_element_type=f32)

    grid_spec = pltpu.PrefetchScalarGridSpec(
        num_scalar_prefetch=1,
        grid=(N_BLK,),
        in_specs=[
            pl.BlockSpec((BLK_M, C), lambda m, be: (m, 0)),
            pl.BlockSpec((1, C, H), lambda m, be: (be[m], 0, 0)),
            pl.BlockSpec((1, C, H), lambda m, be: (be[m], 0, 0)),
            pl.BlockSpec((1, H, C), lambda m, be: (be[m], 0, 0)),
        ],
        out_specs=pl.BlockSpec((BLK_M, C), lambda m, be: (m, 0)),
    )
    y_sorted = pl.pallas_call(
        _expert_body,
        grid_spec=grid_spec,
        out_shape=jax.ShapeDtypeStruct((N_BUF, C), f32),
    )(blk_expert, x_sorted, W1, W3, W2)

    pos01 = jnp.concatenate([pos0, pos1])
    g = _gather_rows(y_sorted, pos01)

    def _shared_body(x_ref, ws1_ref, ws3_ref, ws2_ref, o_ref):
        xb = x_ref[...]
        h1 = jnp.dot(xb, ws1_ref[...], preferred_element_type=f32)
        h3 = jnp.dot(xb, ws3_ref[...], preferred_element_type=f32)
        glu = h1 * _sigmoid(h1) * h3
        o_ref[...] = jnp.dot(glu, ws2_ref[...], preferred_element_type=f32)

    shared_out = pl.pallas_call(
        _shared_body,
        grid=(S // TOK_BLK,),
        in_specs=[
            pl.BlockSpec((TOK_BLK, C), lambda m: (m, 0)),
            pl.BlockSpec((C, H), lambda m: (0, 0)),
            pl.BlockSpec((C, H), lambda m: (0, 0)),
            pl.BlockSpec((H, C), lambda m: (0, 0)),
        ],
        out_specs=pl.BlockSpec((TOK_BLK, C), lambda m: (m, 0)),
        out_shape=jax.ShapeDtypeStruct((S, C), f32),
    )(x_flat, Ws1, Ws3, Ws2)

    def _combine_body(sh_ref, g0_ref, g1_ref, wf_ref, o_ref):
        wf = wf_ref[...]
        o_ref[...] = (sh_ref[...] + wf[:, 0:1] * g0_ref[...]
                      + wf[:, 1:2] * g1_ref[...])

    out = pl.pallas_call(
        _combine_body,
        grid=(S // TOK_BLK,),
        in_specs=[
            pl.BlockSpec((TOK_BLK, C), lambda m: (m, 0)),
            pl.BlockSpec((TOK_BLK, C), lambda m: (m, 0)),
            pl.BlockSpec((TOK_BLK, C), lambda m: (m + S // TOK_BLK, 0)),
            pl.BlockSpec((TOK_BLK, LANE), lambda m: (m, 0)),
        ],
        out_specs=pl.BlockSpec((TOK_BLK, C), lambda m: (m, 0)),
        out_shape=jax.ShapeDtypeStruct((S, C), f32),
    )(shared_out, g, g, w_pad)

    return out.reshape(Bb, Tt, C), aux_loss
